# baseline (device time: 44707 ns/iter reference)
import jax
import jax.numpy as jnp
from jax import lax
from jax.experimental import pallas as pl
from jax.experimental.pallas import tpu as pltpu

N_DEV = 4
SQ = 256
D_MODEL = 1024
HQ = 8
DH = 128
BLK = 64
QB = SQ // BLK
T = 16
SCALE = 0.08838834764831843


def kernel(x, Wq, K_ext, V_ext, Wo):
    K5 = K_ext.reshape(T, QB, BLK, HQ, DH)
    V5 = V_ext.reshape(T, QB, BLK, HQ, DH)
    x2 = x.reshape(SQ, D_MODEL)

    def body(x_ref, wq_ref, k_ref, v_ref, wo_ref, out_ref,
             o_comm, l_comm, ksub, vsub,
             cp_sems, send_o, recv_o, send_l, recv_l):
        me = lax.axis_index("i")

        barrier = pltpu.get_barrier_semaphore()
        for k in range(1, N_DEV):
            pl.semaphore_signal(
                barrier, inc=1,
                device_id=((me + k) % N_DEV,),
                device_id_type=pl.DeviceIdType.MESH)
        pl.semaphore_wait(barrier, N_DEV - 1)

        q_bf = (jnp.dot(
            x_ref[...].astype(jnp.bfloat16),
            wq_ref[...].astype(jnp.bfloat16),
            preferred_element_type=jnp.float32,
        ) * SCALE).astype(jnp.bfloat16)

        tiles = [(h, qb) for h in range(HQ) for qb in range(QB)]

        def start_copy(i):
            h, qb = tiles[i]
            buf = i % 2
            ck = pltpu.make_async_copy(
                k_ref.at[:, qb, :, h, :], ksub.at[buf], cp_sems.at[buf, 0])
            cv = pltpu.make_async_copy(
                v_ref.at[:, qb, :, h, :], vsub.at[buf], cp_sems.at[buf, 1])
            ck.start()
            cv.start()
            return ck, cv

        def send_head(h):
            rs = []
            for k in range(1, N_DEV):
                r = pltpu.make_async_remote_copy(
                    src_ref=o_comm.at[0, h], dst_ref=o_comm.at[k, h],
                    send_sem=send_o.at[k - 1, h], recv_sem=recv_o.at[k - 1, h],
                    device_id=((me + k) % N_DEV,),
                    device_id_type=pl.DeviceIdType.MESH)
                r.start()
                rs.append(r)
            return rs

        inflight = {0: start_copy(0)}
        l_cols, o_run, o_rdmas = [], [], []
        for h in range(HQ):
            l_qbs, o_qbs = [], []
            for qb in range(QB):
                i = h * QB + qb
                buf = i % 2
                if i + 1 < len(tiles):
                    inflight[i + 1] = start_copy(i + 1)
                ck, cv = inflight.pop(i)
                ck.wait()
                cv.wait()
                k = ksub[buf].reshape(T * BLK, DH).astype(jnp.bfloat16)
                v = vsub[buf].reshape(T * BLK, DH).astype(jnp.bfloat16)
                q = q_bf[qb * BLK:(qb + 1) * BLK, h * DH:(h + 1) * DH]
                s = lax.dot_general(
                    q, k, (((1,), (1,)), ((), ())),
                    preferred_element_type=jnp.float32)
                p = jnp.exp(s)
                l_qbs.append(jnp.sum(p, axis=1, keepdims=True))
                o_qbs.append(lax.dot_general(
                    p.astype(jnp.bfloat16), v, (((1,), (0,)), ((), ())),
                    preferred_element_type=jnp.float32))
            l_cols.append(jnp.concatenate(l_qbs, axis=0))
            oh = jnp.concatenate(o_qbs, axis=0)
            o_run.append(oh)
            o_comm[0, h] = oh.astype(jnp.bfloat16)
            o_rdmas.append(send_head(h))
        l_run = jnp.concatenate(l_cols, axis=1)
        l_comm[0] = l_run
        l_rdmas = []
        for k in range(1, N_DEV):
            r = pltpu.make_async_remote_copy(
                src_ref=l_comm.at[0], dst_ref=l_comm.at[k],
                send_sem=send_l.at[k - 1], recv_sem=recv_l.at[k - 1],
                device_id=((me + k) % N_DEV,),
                device_id_type=pl.DeviceIdType.MESH)
            r.start()
            l_rdmas.append(r)

        for k in (1, 3, 2):
            for h in range(HQ):
                o_rdmas[h][k - 1].wait_recv()
                o_run[h] = o_run[h] + o_comm[k, h].astype(jnp.float32)
            l_rdmas[k - 1].wait_recv()
            l_run = l_run + l_comm[k]

        wo = wo_ref[...].astype(jnp.bfloat16)
        acc = jnp.zeros((SQ, D_MODEL), jnp.float32)
        for h in range(HQ):
            ctx_h = (o_run[h] / l_run[:, h:h + 1]).astype(jnp.bfloat16)
            acc = acc + lax.dot_general(
                ctx_h, wo[h * DH:(h + 1) * DH, :],
                (((1,), (0,)), ((), ())),
                preferred_element_type=jnp.float32)
        out_ref[0] = acc

        for rs in o_rdmas:
            for r in rs:
                r.wait_send()
        for r in l_rdmas:
            r.wait_send()

    return pl.pallas_call(
        body,
        out_shape=jax.ShapeDtypeStruct((1, SQ, D_MODEL), jnp.float32),
        in_specs=[
            pl.BlockSpec(memory_space=pltpu.VMEM),
            pl.BlockSpec(memory_space=pltpu.VMEM),
            pl.BlockSpec(memory_space=pltpu.MemorySpace.HBM),
            pl.BlockSpec(memory_space=pltpu.MemorySpace.HBM),
            pl.BlockSpec(memory_space=pltpu.VMEM),
        ],
        out_specs=pl.BlockSpec(memory_space=pltpu.VMEM),
        scratch_shapes=[
            pltpu.VMEM((N_DEV, HQ, SQ, DH), jnp.bfloat16),
            pltpu.VMEM((N_DEV, SQ, HQ), jnp.float32),
            pltpu.VMEM((2, T, BLK, DH), jnp.float32),
            pltpu.VMEM((2, T, BLK, DH), jnp.float32),
            pltpu.SemaphoreType.DMA((2, 2)),
            pltpu.SemaphoreType.DMA((3, HQ)),
            pltpu.SemaphoreType.DMA((3, HQ)),
            pltpu.SemaphoreType.DMA((3,)),
            pltpu.SemaphoreType.DMA((3,)),
        ],
        compiler_params=pltpu.CompilerParams(collective_id=0),
    )(x2, Wq, K5, V5, Wo)


# device time: 32492 ns/iter; 1.3759x vs baseline; 1.3759x over previous
import jax
import jax.numpy as jnp
from jax import lax
from jax.experimental import pallas as pl
from jax.experimental.pallas import tpu as pltpu

N_DEV = 4
SQ = 256
D_MODEL = 1024
HQ = 8
DH = 128
BLK = 64
QB = SQ // BLK
T = 16
SCALE = 0.08838834764831843
DEPTH = 3
NBUF = DEPTH + 1


def kernel(x, Wq, K_ext, V_ext, Wo):
    K5 = K_ext.reshape(T, QB, BLK, HQ, DH)
    V5 = V_ext.reshape(T, QB, BLK, HQ, DH)
    x2 = x.reshape(SQ, D_MODEL)

    def body(x_ref, wq_ref, k_ref, v_ref, wo_ref, out_ref,
             o_comm, l_comm, ksub, vsub,
             cp_sems, send_o, recv_o, send_l, recv_l):
        me = lax.axis_index("i")

        tiles = [(h, qb) for h in range(HQ) for qb in range(QB)]

        def start_copy(i):
            h, qb = tiles[i]
            buf = i % NBUF
            ck = pltpu.make_async_copy(
                k_ref.at[:, qb, :, h, :], ksub.at[buf], cp_sems.at[buf, 0])
            cv = pltpu.make_async_copy(
                v_ref.at[:, qb, :, h, :], vsub.at[buf], cp_sems.at[buf, 1])
            ck.start()
            cv.start()
            return ck, cv

        inflight = {j: start_copy(j) for j in range(DEPTH)}

        barrier = pltpu.get_barrier_semaphore()
        for k in range(1, N_DEV):
            pl.semaphore_signal(
                barrier, inc=1,
                device_id=((me + k) % N_DEV,),
                device_id_type=pl.DeviceIdType.MESH)
        pl.semaphore_wait(barrier, N_DEV - 1)

        q_bf = (jnp.dot(
            x_ref[...].astype(jnp.bfloat16),
            wq_ref[...].astype(jnp.bfloat16),
            preferred_element_type=jnp.float32,
        ) * SCALE).astype(jnp.bfloat16)

        def send_head(h):
            rs = []
            for k in range(1, N_DEV):
                r = pltpu.make_async_remote_copy(
                    src_ref=o_comm.at[0, h], dst_ref=o_comm.at[k, h],
                    send_sem=send_o.at[k - 1, h], recv_sem=recv_o.at[k - 1, h],
                    device_id=((me + k) % N_DEV,),
                    device_id_type=pl.DeviceIdType.MESH)
                r.start()
                rs.append(r)
            return rs

        l_cols, o_run, o_rdmas = [], [], []
        for h in range(HQ):
            l_qbs, o_qbs = [], []
            for qb in range(QB):
                i = h * QB + qb
                buf = i % NBUF
                if i + DEPTH < len(tiles):
                    inflight[i + DEPTH] = start_copy(i + DEPTH)
                ck, cv = inflight.pop(i)
                ck.wait()
                cv.wait()
                k = ksub[buf].reshape(T * BLK, DH).astype(jnp.bfloat16)
                v = vsub[buf].reshape(T * BLK, DH).astype(jnp.bfloat16)
                q = q_bf[qb * BLK:(qb + 1) * BLK, h * DH:(h + 1) * DH]
                s = lax.dot_general(
                    q, k, (((1,), (1,)), ((), ())),
                    preferred_element_type=jnp.float32)
                p = jnp.exp(s)
                l_qbs.append(jnp.sum(p, axis=1, keepdims=True))
                o_qbs.append(lax.dot_general(
                    p.astype(jnp.bfloat16), v, (((1,), (0,)), ((), ())),
                    preferred_element_type=jnp.float32))
            l_cols.append(jnp.concatenate(l_qbs, axis=0))
            oh = jnp.concatenate(o_qbs, axis=0)
            o_run.append(oh)
            o_comm[0, h] = oh.astype(jnp.bfloat16)
            o_rdmas.append(send_head(h))
        l_run = jnp.concatenate(l_cols, axis=1)
        l_comm[0] = l_run
        l_rdmas = []
        for k in range(1, N_DEV):
            r = pltpu.make_async_remote_copy(
                src_ref=l_comm.at[0], dst_ref=l_comm.at[k],
                send_sem=send_l.at[k - 1], recv_sem=recv_l.at[k - 1],
                device_id=((me + k) % N_DEV,),
                device_id_type=pl.DeviceIdType.MESH)
            r.start()
            l_rdmas.append(r)

        for k in (1, 3, 2):
            for h in range(HQ):
                o_rdmas[h][k - 1].wait_recv()
                o_run[h] = o_run[h] + o_comm[k, h].astype(jnp.float32)
            l_rdmas[k - 1].wait_recv()
            l_run = l_run + l_comm[k]

        wo = wo_ref[...].astype(jnp.bfloat16)
        acc = jnp.zeros((SQ, D_MODEL), jnp.float32)
        for h in range(HQ):
            ctx_h = (o_run[h] / l_run[:, h:h + 1]).astype(jnp.bfloat16)
            acc = acc + lax.dot_general(
                ctx_h, wo[h * DH:(h + 1) * DH, :],
                (((1,), (0,)), ((), ())),
                preferred_element_type=jnp.float32)
        out_ref[0] = acc

        for rs in o_rdmas:
            for r in rs:
                r.wait_send()
        for r in l_rdmas:
            r.wait_send()

    return pl.pallas_call(
        body,
        out_shape=jax.ShapeDtypeStruct((1, SQ, D_MODEL), jnp.float32),
        in_specs=[
            pl.BlockSpec(memory_space=pltpu.VMEM),
            pl.BlockSpec(memory_space=pltpu.VMEM),
            pl.BlockSpec(memory_space=pltpu.MemorySpace.HBM),
            pl.BlockSpec(memory_space=pltpu.MemorySpace.HBM),
            pl.BlockSpec(memory_space=pltpu.VMEM),
        ],
        out_specs=pl.BlockSpec(memory_space=pltpu.VMEM),
        scratch_shapes=[
            pltpu.VMEM((N_DEV, HQ, SQ, DH), jnp.bfloat16),
            pltpu.VMEM((N_DEV, SQ, HQ), jnp.float32),
            pltpu.VMEM((NBUF, T, BLK, DH), jnp.float32),
            pltpu.VMEM((NBUF, T, BLK, DH), jnp.float32),
            pltpu.SemaphoreType.DMA((NBUF, 2)),
            pltpu.SemaphoreType.DMA((3, HQ)),
            pltpu.SemaphoreType.DMA((3, HQ)),
            pltpu.SemaphoreType.DMA((3,)),
            pltpu.SemaphoreType.DMA((3,)),
        ],
        compiler_params=pltpu.CompilerParams(collective_id=0),
    )(x2, Wq, K5, V5, Wo)
